# baseline (speedup 1.0000x reference)
import functools

import jax
import jax.numpy as jnp
from jax import lax
from jax.experimental import pallas as pl
from jax.experimental.pallas import tpu as pltpu
from jax.experimental.pallas import tpu_sc as plsc

_CHUNK = 128
_L = 16


@functools.lru_cache(maxsize=None)
def _build(B, V, D):
    info = plsc.get_sparse_core_info()
    NC, NS = info.num_cores, info.num_subcores
    NW = NC * NS
    assert B % (NW * _CHUNK) == 0, (B, NW)
    b_per_w = B // NW
    n_chunks = b_per_w // _CHUNK
    rg = _CHUNK // D

    mesh = plsc.VectorSubcoreMesh(core_axis_name="c", subcore_axis_name="s")

    @functools.partial(
        pl.kernel,
        mesh=mesh,
        compiler_params=pltpu.CompilerParams(needs_layout_passes=False),
        out_type=jax.ShapeDtypeStruct((B * D // _CHUNK, _CHUNK), jnp.float32),
        scratch_types=[
            pltpu.VMEM((b_per_w,), jnp.int32),
            pltpu.VMEM((n_chunks, _CHUNK), jnp.int32),
            pltpu.VMEM((_CHUNK, _CHUNK), jnp.float32),
            pltpu.VMEM((b_per_w * D // _CHUNK, _CHUNK), jnp.float32),
            pltpu.SemaphoreType.DMA,
        ],
    )
    def gather_kernel(ids_hbm, table_hbm, out_hbm, idx_v, grp_v, ext_v, out_v,
                      sem):
        wid = lax.axis_index("s") * NC + lax.axis_index("c")
        base = wid * b_per_w
        pltpu.sync_copy(ids_hbm.at[pl.ds(base, b_per_w)], idx_v)
        lanes = lax.iota(jnp.int32, _L)

        def grp_block(g, carry):
            id16 = idx_v[pl.ds(g * _L, _L)]
            k = g * _L + lanes
            plsc.store_scatter(
                grp_v,
                [lax.shift_right_logical(k, 7), lax.bitwise_and(k, 127)],
                lax.shift_right_logical(id16, 2),
            )
            return carry

        lax.fori_loop(0, b_per_w // _L, grp_block, 0)

        for q in range(n_chunks):
            pltpu.async_copy(table_hbm.at[grp_v.at[q]], ext_v, sem).wait()
            for g in range(_CHUNK // _L):
                pos = g * _L + lanes
                id16 = idx_v[pl.ds(q * _CHUNK + g * _L, _L)]
                col0 = lax.bitwise_and(id16, rg - 1) * D
                w_base = (q * _CHUNK + pos) * D
                for c in range(D):
                    val = plsc.load_gather(ext_v, [pos, col0 + c])
                    w = w_base + c
                    plsc.store_scatter(
                        out_v,
                        [
                            lax.shift_right_logical(w, 7),
                            lax.bitwise_and(w, 127),
                        ],
                        val,
                    )
        pltpu.sync_copy(
            out_v, out_hbm.at[pl.ds(wid * (b_per_w * D // _CHUNK),
                                    b_per_w * D // _CHUNK)]
        )

    return gather_kernel


def kernel(phoneme_ids, table):
    (B,) = phoneme_ids.shape
    V, D = table.shape
    fn = _build(B, V, D)
    table128 = table.reshape(V * D // _CHUNK, _CHUNK)
    out = fn(phoneme_ids.astype(jnp.int32), table128)
    return out.reshape(B, D)
